# baseline (speedup 1.0000x reference)
import functools

import jax
import jax.numpy as jnp
from jax import lax
from jax.experimental import pallas as pl
from jax.experimental import layout as jex_layout
from jax.experimental.pallas import tpu as pltpu
from jax.experimental.pallas import tpu_sc as plsc

NC = 2
NS = 16
NW = NC * NS


@functools.partial(jax.jit, static_argnames=("h", "v", "d"))
def _embed_sc(xt, weight, h, v, d):
    b = xt.shape[1]
    bw = b // NW

    mesh = plsc.VectorSubcoreMesh(core_axis_name="c", subcore_axis_name="s")

    @functools.partial(
        pl.kernel,
        out_type=jax.ShapeDtypeStruct((h, d, b), jnp.float32),
        mesh=mesh,
        compiler_params=pltpu.CompilerParams(
            use_tc_tiling_on_sc=False, needs_layout_passes=False
        ),
        scratch_types=[
            pltpu.VMEM((h, bw), jnp.int32),
            pltpu.VMEM((bw, d), jnp.float32),
            pltpu.VMEM((bw, d), jnp.float32),
            pltpu.VMEM((d, bw), jnp.float32),
            pltpu.VMEM((d, bw), jnp.float32),
            pltpu.SemaphoreType.DMA,
            pltpu.SemaphoreType.DMA,
            pltpu.SemaphoreType.DMA,
            pltpu.SemaphoreType.DMA,
        ],
    )
    def body(xt_hbm, table_hbm, out_hbm, idx_v, g0, g1, t0, t1, gs0, gs1, ws0, ws1):
        gbuf = (g0, g1)
        tbuf = (t0, t1)
        gsem = (gs0, gs1)
        wsem = (ws0, ws1)
        wid = lax.axis_index("s") * NC + lax.axis_index("c")
        b0 = wid * bw
        pltpu.sync_copy(xt_hbm.at[:, pl.ds(b0, bw)], idx_v)

        iota = lax.iota(jnp.int32, 16)
        rvecs = [v16 * 16 + iota for v16 in range(bw // 16)]

        for s in range(2):
            pltpu.async_copy(table_hbm.at[idx_v.at[s]], gbuf[s], gsem[s])

        def block(g, _):
            for s in range(2):
                hh = g * 2 + s
                pltpu.make_async_copy(
                    table_hbm.at[idx_v.at[0]], gbuf[s], gsem[s]
                ).wait()

                @pl.when(g > 0)
                def _():
                    pltpu.make_async_copy(
                        tbuf[s], out_hbm.at[0, :, pl.ds(b0, bw)], wsem[s]
                    ).wait()

                for c in range(d):
                    cvec = jnp.full((16,), c, jnp.int32)
                    for v16 in range(bw // 16):
                        val = plsc.load_gather(gbuf[s], [rvecs[v16], cvec])
                        tbuf[s][c, pl.ds(v16 * 16, 16)] = val

                pltpu.async_copy(
                    tbuf[s], out_hbm.at[hh, :, pl.ds(b0, bw)], wsem[s]
                )

                @pl.when(g < h // 2 - 1)
                def _():
                    pltpu.async_copy(
                        table_hbm.at[idx_v.at[hh + 2]], gbuf[s], gsem[s]
                    )

            return _

        lax.fori_loop(0, h // 2, block, None)
        for s in range(2):
            pltpu.make_async_copy(
                tbuf[s], out_hbm.at[0, :, pl.ds(b0, bw)], wsem[s]
            ).wait()

    return body(xt, weight)


def kernel(x, weight):
    b, h = x.shape
    v, d = weight.shape
    xt = x.T.astype(jnp.int32)
    wt = jex_layout.with_layout_constraint(
        weight, jex_layout.Layout((1, 0), tiling=((8,),))
    )
    out3 = _embed_sc(xt, wt, h, v, d)
    return out3.transpose(2, 0, 1)


# speedup vs baseline: 1.2260x; 1.2260x over previous
import functools

import jax
import jax.numpy as jnp
from jax import lax
from jax.experimental import pallas as pl
from jax.experimental import layout as jex_layout
from jax.experimental.pallas import tpu as pltpu
from jax.experimental.pallas import tpu_sc as plsc

NC = 2
NS = 16
NW = NC * NS


@functools.partial(jax.jit, static_argnames=("h", "v", "d"))
def _embed_sc(xt, weight, h, v, d):
    b = xt.shape[1]
    bw = b // NW

    mesh = plsc.VectorSubcoreMesh(core_axis_name="c", subcore_axis_name="s")

    @functools.partial(
        pl.kernel,
        out_type=jax.ShapeDtypeStruct((h, d, b), jnp.float32),
        mesh=mesh,
        compiler_params=pltpu.CompilerParams(
            use_tc_tiling_on_sc=False, needs_layout_passes=False
        ),
        scratch_types=[
            pltpu.VMEM((h, bw), jnp.int32),
            pltpu.VMEM((bw, d), jnp.float32),
            pltpu.VMEM((bw, d), jnp.float32),
            pltpu.VMEM((d, bw), jnp.float32),
            pltpu.VMEM((d, bw), jnp.float32),
            pltpu.SemaphoreType.DMA,
            pltpu.SemaphoreType.DMA,
            pltpu.SemaphoreType.DMA,
            pltpu.SemaphoreType.DMA,
        ],
    )
    def body(xt_hbm, table_hbm, out_hbm, idx_v, g0, g1, t0, t1, gs0, gs1, ws0, ws1):
        gbuf = (g0, g1)
        tbuf = (t0, t1)
        gsem = (gs0, gs1)
        wsem = (ws0, ws1)
        wid = lax.axis_index("s") * NC + lax.axis_index("c")
        b0 = wid * bw
        pltpu.sync_copy(xt_hbm.at[:, pl.ds(b0, bw)], idx_v)

        iota = lax.iota(jnp.int32, 16)
        rvecs = [v16 * 16 + iota for v16 in range(bw // 16)]

        for s in range(2):
            pltpu.async_copy(table_hbm.at[idx_v.at[s]], gbuf[s], gsem[s])

        def block(g, _):
            for s in range(2):
                hh = g * 2 + s
                pltpu.make_async_copy(
                    table_hbm.at[idx_v.at[0]], gbuf[s], gsem[s]
                ).wait()

                @pl.when(g > 0)
                def _():
                    pltpu.make_async_copy(
                        tbuf[s], out_hbm.at[0, :, pl.ds(b0, bw)], wsem[s]
                    ).wait()

                for c in range(d):
                    cvec = jnp.full((16,), c, jnp.int32)
                    vals = [
                        plsc.load_gather(gbuf[s], [rvecs[v16], cvec])
                        for v16 in range(bw // 16)
                    ]
                    for v16 in range(bw // 16):
                        tbuf[s][c, pl.ds(v16 * 16, 16)] = vals[v16]

                pltpu.async_copy(
                    tbuf[s], out_hbm.at[hh, :, pl.ds(b0, bw)], wsem[s]
                )

                @pl.when(g < h // 2 - 1)
                def _():
                    pltpu.async_copy(
                        table_hbm.at[idx_v.at[hh + 2]], gbuf[s], gsem[s]
                    )

            return _

        lax.fori_loop(0, h // 2, block, None)
        for s in range(2):
            pltpu.make_async_copy(
                tbuf[s], out_hbm.at[0, :, pl.ds(b0, bw)], wsem[s]
            ).wait()

    return body(xt, weight)


def kernel(x, weight):
    b, h = x.shape
    v, d = weight.shape
    xt = x.T.astype(jnp.int32)
    wt = jex_layout.with_layout_constraint(
        weight, jex_layout.Layout((1, 0), tiling=((8,),))
    )
    out3 = _embed_sc(xt, wt, h, v, d)
    return out3.transpose(2, 0, 1)


# speedup vs baseline: 1.2663x; 1.0329x over previous
import functools

import jax
import jax.numpy as jnp
from jax import lax
from jax.experimental import pallas as pl
from jax.experimental import layout as jex_layout
from jax.experimental.pallas import tpu as pltpu
from jax.experimental.pallas import tpu_sc as plsc

NC = 2
NS = 16
NW = NC * NS


@functools.partial(jax.jit, static_argnames=("h", "v", "d"))
def _embed_sc(xt, weight, h, v, d):
    b = xt.shape[1]
    bw = b // NW

    mesh = plsc.VectorSubcoreMesh(core_axis_name="c", subcore_axis_name="s")

    @functools.partial(
        pl.kernel,
        out_type=jax.ShapeDtypeStruct((h, NW, d, b // NW), jnp.float32),
        mesh=mesh,
        compiler_params=pltpu.CompilerParams(
            use_tc_tiling_on_sc=False, needs_layout_passes=False
        ),
        scratch_types=[
            pltpu.VMEM((h, bw), jnp.int32),
            pltpu.VMEM((bw, d), jnp.float32),
            pltpu.VMEM((bw, d), jnp.float32),
            pltpu.VMEM((d, bw), jnp.float32),
            pltpu.VMEM((d, bw), jnp.float32),
            pltpu.SemaphoreType.DMA,
            pltpu.SemaphoreType.DMA,
            pltpu.SemaphoreType.DMA,
            pltpu.SemaphoreType.DMA,
        ],
    )
    def body(xt_hbm, table_hbm, out_hbm, idx_v, g0, g1, t0, t1, gs0, gs1, ws0, ws1):
        gbuf = (g0, g1)
        tbuf = (t0, t1)
        gsem = (gs0, gs1)
        wsem = (ws0, ws1)
        wid = lax.axis_index("s") * NC + lax.axis_index("c")
        b0 = wid * bw
        pltpu.sync_copy(xt_hbm.at[:, pl.ds(b0, bw)], idx_v)

        iota = lax.iota(jnp.int32, 16)
        rvecs = [v16 * 16 + iota for v16 in range(bw // 16)]

        for s in range(2):
            pltpu.async_copy(table_hbm.at[idx_v.at[s]], gbuf[s], gsem[s])

        def block(g, _):
            for s in range(2):
                hh = g * 2 + s
                pltpu.make_async_copy(
                    table_hbm.at[idx_v.at[0]], gbuf[s], gsem[s]
                ).wait()

                @pl.when(g > 0)
                def _():
                    pltpu.make_async_copy(
                        tbuf[s], out_hbm.at[0, wid], wsem[s]
                    ).wait()

                for c in range(d):
                    cvec = jnp.full((16,), c, jnp.int32)
                    vals = [
                        plsc.load_gather(gbuf[s], [rvecs[v16], cvec])
                        for v16 in range(bw // 16)
                    ]
                    for v16 in range(bw // 16):
                        tbuf[s][c, pl.ds(v16 * 16, 16)] = vals[v16]

                pltpu.async_copy(tbuf[s], out_hbm.at[hh, wid], wsem[s])

                @pl.when(g < h // 2 - 1)
                def _():
                    pltpu.async_copy(
                        table_hbm.at[idx_v.at[hh + 2]], gbuf[s], gsem[s]
                    )

            return _

        lax.fori_loop(0, h // 2, block, None)
        for s in range(2):
            pltpu.make_async_copy(tbuf[s], out_hbm.at[0, wid], wsem[s]).wait()

    return body(xt, weight)


def kernel(x, weight):
    b, h = x.shape
    v, d = weight.shape
    xt = x.T.astype(jnp.int32)
    wt = jex_layout.with_layout_constraint(
        weight, jex_layout.Layout((1, 0), tiling=((8,),))
    )
    out4 = _embed_sc(xt, wt, h, v, d)
    return out4.transpose(1, 3, 0, 2).reshape(b, h, d)
